# baseline (speedup 1.0000x reference)
import functools

import jax
import jax.numpy as jnp
from jax import lax
from jax.experimental import pallas as pl
from jax.experimental.pallas import tpu as pltpu
from jax.experimental.pallas import tpu_sc as plsc

N_NODES = 10000
N_EDGES = 320000
D_IN = 128
D_HID = 16
W8 = 8

NC = 2
NS = 16
NW = NC * NS
EPW = N_EDGES // NW
CH = 2000
NCH = EPW // CH
SL_A = 624
SL_B = N_NODES - 15 * SL_A


def _tc_linear2(x, wl, wr):
    def body(x_ref, wl_ref, wr_ref, y_ref, p_ref):
        xv = x_ref[...]
        y_ref[...] = jnp.dot(xv, wl_ref[...], preferred_element_type=jnp.float32,
                             precision=lax.Precision.HIGHEST)
        p_ref[...] = jnp.dot(xv, wr_ref[...], preferred_element_type=jnp.float32,
                             precision=lax.Precision.HIGHEST)

    n = x.shape[0]
    return pl.pallas_call(
        body,
        out_shape=[
            jax.ShapeDtypeStruct((n, wl.shape[1]), jnp.float32),
            jax.ShapeDtypeStruct((n, wr.shape[1]), jnp.float32),
        ],
    )(x, wl, wr)


def _sc_aggregate1(src, dst, y1, zeros16, zeros8, ones8):
    mesh = plsc.VectorSubcoreMesh(core_axis_name="c", subcore_axis_name="s")

    @functools.partial(
        pl.kernel,
        out_type=[
            jax.ShapeDtypeStruct((NC * N_NODES, D_HID), jnp.float32),
            jax.ShapeDtypeStruct((NC * N_NODES, W8), jnp.float32),
        ],
        mesh=mesh,
        compiler_params=pltpu.CompilerParams(use_tc_tiling_on_sc=False),
        scratch_types=[
            pltpu.VMEM_SHARED((N_NODES, D_HID), jnp.float32),
            pltpu.VMEM_SHARED((N_NODES, W8), jnp.float32),
            pltpu.VMEM((CH,), jnp.int32),
            pltpu.VMEM((CH,), jnp.int32),
            pltpu.VMEM((CH, D_HID), jnp.float32),
            pltpu.VMEM((CH, W8), jnp.float32),
            pltpu.VMEM((SL_B, D_HID), jnp.float32),
            pltpu.VMEM((SL_B, W8), jnp.float32),
            pltpu.SemaphoreType.DMA,
        ],
    )
    def k(src_hbm, dst_hbm, y1_hbm, z16_hbm, z8_hbm, ones_hbm,
          agg_out, deg_out, agg_sh, deg_sh,
          src_v, dst_v, rows_v, ones_v, st16_v, st8_v, sem):
        c = lax.axis_index("c")
        s = lax.axis_index("s")
        wid = s * NC + c

        pltpu.sync_copy(ones_hbm, ones_v)

        @pl.when(s < 15)
        def _():
            n0 = s * SL_A
            pltpu.sync_copy(z16_hbm.at[pl.ds(0, SL_A)], st16_v.at[pl.ds(0, SL_A)])
            pltpu.sync_copy(st16_v.at[pl.ds(0, SL_A)], agg_sh.at[pl.ds(n0, SL_A)])
            pltpu.sync_copy(z8_hbm.at[pl.ds(0, SL_A)], st8_v.at[pl.ds(0, SL_A)])
            pltpu.sync_copy(st8_v.at[pl.ds(0, SL_A)], deg_sh.at[pl.ds(n0, SL_A)])

        @pl.when(s == 15)
        def _():
            pltpu.sync_copy(z16_hbm, st16_v)
            pltpu.sync_copy(st16_v, agg_sh.at[pl.ds(15 * SL_A, SL_B)])
            pltpu.sync_copy(z8_hbm, st8_v)
            pltpu.sync_copy(st8_v, deg_sh.at[pl.ds(15 * SL_A, SL_B)])

        plsc.subcore_barrier()

        base = wid * EPW

        def eloop(j, _):
            off = pl.multiple_of(base + j * CH, 8)
            pltpu.sync_copy(src_hbm.at[pl.ds(off, CH)], src_v)
            pltpu.sync_copy(dst_hbm.at[pl.ds(off, CH)], dst_v)
            pltpu.async_copy(y1_hbm.at[src_v], rows_v, sem).wait()
            pltpu.sync_copy(rows_v, agg_sh.at[dst_v], add=True)
            pltpu.sync_copy(ones_v, deg_sh.at[dst_v], add=True)
            return 0

        lax.fori_loop(0, NCH, eloop, 0)
        plsc.subcore_barrier()

        @pl.when(s < 15)
        def _():
            n0 = s * SL_A
            r0 = c * N_NODES + n0
            pltpu.sync_copy(agg_sh.at[pl.ds(n0, SL_A)], st16_v.at[pl.ds(0, SL_A)])
            pltpu.sync_copy(st16_v.at[pl.ds(0, SL_A)], agg_out.at[pl.ds(r0, SL_A)])
            pltpu.sync_copy(deg_sh.at[pl.ds(n0, SL_A)], st8_v.at[pl.ds(0, SL_A)])
            pltpu.sync_copy(st8_v.at[pl.ds(0, SL_A)], deg_out.at[pl.ds(r0, SL_A)])

        @pl.when(s == 15)
        def _():
            n0 = 15 * SL_A
            r0 = c * N_NODES + n0
            pltpu.sync_copy(agg_sh.at[pl.ds(n0, SL_B)], st16_v)
            pltpu.sync_copy(st16_v, agg_out.at[pl.ds(r0, SL_B)])
            pltpu.sync_copy(deg_sh.at[pl.ds(n0, SL_B)], st8_v)
            pltpu.sync_copy(st8_v, deg_out.at[pl.ds(r0, SL_B)])

    return k(src, dst, y1, zeros16, zeros8, ones8)


def _sc_aggregate2(src, dst, y2b, zeros8):
    mesh = plsc.VectorSubcoreMesh(core_axis_name="c", subcore_axis_name="s")

    @functools.partial(
        pl.kernel,
        out_type=jax.ShapeDtypeStruct((NC * N_NODES, W8), jnp.float32),
        mesh=mesh,
        compiler_params=pltpu.CompilerParams(use_tc_tiling_on_sc=False),
        scratch_types=[
            pltpu.VMEM_SHARED((N_NODES, W8), jnp.float32),
            pltpu.VMEM((CH,), jnp.int32),
            pltpu.VMEM((CH,), jnp.int32),
            pltpu.VMEM((CH, W8), jnp.float32),
            pltpu.VMEM((SL_B, W8), jnp.float32),
            pltpu.SemaphoreType.DMA,
        ],
    )
    def k(src_hbm, dst_hbm, y2_hbm, z8_hbm, agg_out, agg_sh,
          src_v, dst_v, rows_v, st8_v, sem):
        c = lax.axis_index("c")
        s = lax.axis_index("s")
        wid = s * NC + c

        @pl.when(s < 15)
        def _():
            pltpu.sync_copy(z8_hbm.at[pl.ds(0, SL_A)], st8_v.at[pl.ds(0, SL_A)])
            pltpu.sync_copy(st8_v.at[pl.ds(0, SL_A)],
                            agg_sh.at[pl.ds(s * SL_A, SL_A)])

        @pl.when(s == 15)
        def _():
            pltpu.sync_copy(z8_hbm, st8_v)
            pltpu.sync_copy(st8_v, agg_sh.at[pl.ds(15 * SL_A, SL_B)])

        plsc.subcore_barrier()

        base = wid * EPW

        def eloop(j, _):
            off = pl.multiple_of(base + j * CH, 8)
            pltpu.sync_copy(src_hbm.at[pl.ds(off, CH)], src_v)
            pltpu.sync_copy(dst_hbm.at[pl.ds(off, CH)], dst_v)
            pltpu.async_copy(y2_hbm.at[src_v], rows_v, sem).wait()
            pltpu.sync_copy(rows_v, agg_sh.at[dst_v], add=True)
            return 0

        lax.fori_loop(0, NCH, eloop, 0)
        plsc.subcore_barrier()

        @pl.when(s < 15)
        def _():
            n0 = s * SL_A
            pltpu.sync_copy(agg_sh.at[pl.ds(n0, SL_A)], st8_v.at[pl.ds(0, SL_A)])
            pltpu.sync_copy(st8_v.at[pl.ds(0, SL_A)],
                            agg_out.at[pl.ds(c * N_NODES + n0, SL_A)])

        @pl.when(s == 15)
        def _():
            n0 = 15 * SL_A
            pltpu.sync_copy(agg_sh.at[pl.ds(n0, SL_B)], st8_v)
            pltpu.sync_copy(st8_v, agg_out.at[pl.ds(c * N_NODES + n0, SL_B)])

    return k(src, dst, y2b, zeros8)


def _tc_layer_mid(aggp, degp, p1, b1l, w2l, w2r, b2l):
    def body(aggp_ref, degp_ref, p1_ref, b1l_ref, w2l_ref, w2r_ref, b2l_ref,
             y2_ref, p2b_ref, degc_ref):
        agg = aggp_ref[0:N_NODES, :] + aggp_ref[N_NODES:2 * N_NODES, :]
        deg = degp_ref[0:N_NODES, 0:1] + degp_ref[N_NODES:2 * N_NODES, 0:1]
        degc = jnp.maximum(deg, 1.0)
        h = jnp.maximum(agg / degc + b1l_ref[...] + p1_ref[...], 0.0)
        y2 = jnp.sum(h * w2l_ref[...], axis=1, keepdims=True)
        y2_ref[...] = jnp.broadcast_to(y2, (N_NODES, W8))
        p2b_ref[...] = jnp.sum(h * w2r_ref[...], axis=1, keepdims=True) + b2l_ref[...]
        degc_ref[...] = degc

    return pl.pallas_call(
        body,
        out_shape=[
            jax.ShapeDtypeStruct((N_NODES, W8), jnp.float32),
            jax.ShapeDtypeStruct((N_NODES, 1), jnp.float32),
            jax.ShapeDtypeStruct((N_NODES, 1), jnp.float32),
        ],
    )(aggp, degp, p1, b1l, w2l, w2r, b2l)


def _tc_final(agg2p, degc, p2b):
    def body(a_ref, d_ref, p_ref, o_ref):
        a = a_ref[0:N_NODES, 0:1] + a_ref[N_NODES:2 * N_NODES, 0:1]
        o_ref[...] = a / d_ref[...] + p_ref[...]

    return pl.pallas_call(
        body,
        out_shape=jax.ShapeDtypeStruct((N_NODES, 1), jnp.float32),
    )(agg2p, degc, p2b)


def kernel(x, edge_index, W1l, b1l, W1r, W2l, b2l, W2r):
    src = edge_index[0].astype(jnp.int32)
    dst = edge_index[1].astype(jnp.int32)

    zeros16 = jnp.zeros((SL_B, D_HID), jnp.float32)
    zeros8 = jnp.zeros((SL_B, W8), jnp.float32)
    ones8 = jnp.ones((CH, W8), jnp.float32)

    y1, p1 = _tc_linear2(x, W1l.T, W1r.T)
    aggp, degp = _sc_aggregate1(src, dst, y1, zeros16, zeros8, ones8)
    y2b, p2b, degc = _tc_layer_mid(
        aggp, degp, p1, b1l.reshape(1, D_HID), W2l, W2r, b2l.reshape(1, 1))
    agg2p = _sc_aggregate2(src, dst, y2b, zeros8)
    out = _tc_final(agg2p, degc, p2b)
    return out


# speedup vs baseline: 1.1053x; 1.1053x over previous
import functools

import jax
import jax.numpy as jnp
from jax import lax
from jax.experimental import pallas as pl
from jax.experimental.pallas import tpu as pltpu
from jax.experimental.pallas import tpu_sc as plsc

N_NODES = 10000
N_EDGES = 320000
D_IN = 128
D_HID = 16
W8 = 8

NC = 2
NS = 16
NW = NC * NS
EPW = N_EDGES // NW
CH = 2000
NCH = EPW // CH
SL_A = 624
SL_B = N_NODES - 15 * SL_A


def _tc_linear2(x, wl, wr):
    def body(x_ref, wl_ref, wr_ref, y_ref, p_ref):
        xv = x_ref[...]
        y_ref[...] = jnp.dot(xv, wl_ref[...], preferred_element_type=jnp.float32,
                             precision=lax.Precision.HIGHEST)
        p_ref[...] = jnp.dot(xv, wr_ref[...], preferred_element_type=jnp.float32,
                             precision=lax.Precision.HIGHEST)

    n = x.shape[0]
    return pl.pallas_call(
        body,
        out_shape=[
            jax.ShapeDtypeStruct((n, wl.shape[1]), jnp.float32),
            jax.ShapeDtypeStruct((n, wr.shape[1]), jnp.float32),
        ],
    )(x, wl, wr)


def _sc_aggregate1(src, dst, y1, zeros16, zeros8, ones8):
    mesh = plsc.VectorSubcoreMesh(core_axis_name="c", subcore_axis_name="s")

    @functools.partial(
        pl.kernel,
        out_type=[
            jax.ShapeDtypeStruct((NC * N_NODES, D_HID), jnp.float32),
            jax.ShapeDtypeStruct((NC * N_NODES, W8), jnp.float32),
        ],
        mesh=mesh,
        compiler_params=pltpu.CompilerParams(use_tc_tiling_on_sc=False),
        scratch_types=[
            pltpu.VMEM_SHARED((N_NODES, D_HID), jnp.float32),
            pltpu.VMEM_SHARED((N_NODES, W8), jnp.float32),
            pltpu.VMEM((CH,), jnp.int32),
            pltpu.VMEM((CH,), jnp.int32),
            pltpu.VMEM((CH, D_HID), jnp.float32),
            pltpu.VMEM((CH,), jnp.int32),
            pltpu.VMEM((CH,), jnp.int32),
            pltpu.VMEM((CH, D_HID), jnp.float32),
            pltpu.VMEM((CH, W8), jnp.float32),
            pltpu.VMEM((SL_B, D_HID), jnp.float32),
            pltpu.VMEM((SL_B, W8), jnp.float32),
            pltpu.SemaphoreType.DMA,
            pltpu.SemaphoreType.DMA,
            pltpu.SemaphoreType.DMA,
            pltpu.SemaphoreType.DMA,
        ],
    )
    def k(src_hbm, dst_hbm, y1_hbm, z16_hbm, z8_hbm, ones_hbm,
          agg_out, deg_out, agg_sh, deg_sh,
          src_v0, dst_v0, rows_v0, src_v1, dst_v1, rows_v1,
          ones_v, st16_v, st8_v, sem_g0, sem_g1, sem_s1, sem_s2):
        c = lax.axis_index("c")
        s = lax.axis_index("s")
        wid = s * NC + c

        pltpu.sync_copy(ones_hbm, ones_v)

        @pl.when(s < 15)
        def _():
            n0 = s * SL_A
            pltpu.sync_copy(z16_hbm.at[pl.ds(0, SL_A)], st16_v.at[pl.ds(0, SL_A)])
            pltpu.sync_copy(st16_v.at[pl.ds(0, SL_A)], agg_sh.at[pl.ds(n0, SL_A)])
            pltpu.sync_copy(z8_hbm.at[pl.ds(0, SL_A)], st8_v.at[pl.ds(0, SL_A)])
            pltpu.sync_copy(st8_v.at[pl.ds(0, SL_A)], deg_sh.at[pl.ds(n0, SL_A)])

        @pl.when(s == 15)
        def _():
            pltpu.sync_copy(z16_hbm, st16_v)
            pltpu.sync_copy(st16_v, agg_sh.at[pl.ds(15 * SL_A, SL_B)])
            pltpu.sync_copy(z8_hbm, st8_v)
            pltpu.sync_copy(st8_v, deg_sh.at[pl.ds(15 * SL_A, SL_B)])

        plsc.subcore_barrier()

        base = wid * EPW
        srcb = [src_v0, src_v1]
        dstb = [dst_v0, dst_v1]
        rowb = [rows_v0, rows_v1]
        semg = [sem_g0, sem_g1]

        def load_idx(j):
            off = pl.multiple_of(base + j * CH, 8)
            pltpu.sync_copy(src_hbm.at[pl.ds(off, CH)], srcb[j % 2])
            pltpu.sync_copy(dst_hbm.at[pl.ds(off, CH)], dstb[j % 2])

        load_idx(0)
        gathers = {0: pltpu.async_copy(y1_hbm.at[srcb[0]], rowb[0], sem_g0)}
        scatters = {}
        for j in range(NCH):
            if j >= 1:
                for d in scatters.pop(j - 1):
                    d.wait()
            if j + 1 < NCH:
                load_idx(j + 1)
            gathers.pop(j).wait()
            scatters[j] = (
                pltpu.async_copy(rowb[j % 2], agg_sh.at[dstb[j % 2]], sem_s1,
                                 add=True),
                pltpu.async_copy(ones_v, deg_sh.at[dstb[j % 2]], sem_s2,
                                 add=True),
            )
            if j + 1 < NCH:
                gathers[j + 1] = pltpu.async_copy(
                    y1_hbm.at[srcb[(j + 1) % 2]], rowb[(j + 1) % 2],
                    semg[(j + 1) % 2])
        for d in scatters.pop(NCH - 1):
            d.wait()
        plsc.subcore_barrier()

        @pl.when(s < 15)
        def _():
            n0 = s * SL_A
            r0 = c * N_NODES + n0
            pltpu.sync_copy(agg_sh.at[pl.ds(n0, SL_A)], st16_v.at[pl.ds(0, SL_A)])
            pltpu.sync_copy(st16_v.at[pl.ds(0, SL_A)], agg_out.at[pl.ds(r0, SL_A)])
            pltpu.sync_copy(deg_sh.at[pl.ds(n0, SL_A)], st8_v.at[pl.ds(0, SL_A)])
            pltpu.sync_copy(st8_v.at[pl.ds(0, SL_A)], deg_out.at[pl.ds(r0, SL_A)])

        @pl.when(s == 15)
        def _():
            n0 = 15 * SL_A
            r0 = c * N_NODES + n0
            pltpu.sync_copy(agg_sh.at[pl.ds(n0, SL_B)], st16_v)
            pltpu.sync_copy(st16_v, agg_out.at[pl.ds(r0, SL_B)])
            pltpu.sync_copy(deg_sh.at[pl.ds(n0, SL_B)], st8_v)
            pltpu.sync_copy(st8_v, deg_out.at[pl.ds(r0, SL_B)])

    return k(src, dst, y1, zeros16, zeros8, ones8)


def _sc_aggregate2(src, dst, y2b, zeros8):
    mesh = plsc.VectorSubcoreMesh(core_axis_name="c", subcore_axis_name="s")

    @functools.partial(
        pl.kernel,
        out_type=jax.ShapeDtypeStruct((NC * N_NODES, W8), jnp.float32),
        mesh=mesh,
        compiler_params=pltpu.CompilerParams(use_tc_tiling_on_sc=False),
        scratch_types=[
            pltpu.VMEM_SHARED((N_NODES, W8), jnp.float32),
            pltpu.VMEM((CH,), jnp.int32),
            pltpu.VMEM((CH,), jnp.int32),
            pltpu.VMEM((CH, W8), jnp.float32),
            pltpu.VMEM((CH,), jnp.int32),
            pltpu.VMEM((CH,), jnp.int32),
            pltpu.VMEM((CH, W8), jnp.float32),
            pltpu.VMEM((SL_B, W8), jnp.float32),
            pltpu.SemaphoreType.DMA,
            pltpu.SemaphoreType.DMA,
            pltpu.SemaphoreType.DMA,
        ],
    )
    def k(src_hbm, dst_hbm, y2_hbm, z8_hbm, agg_out, agg_sh,
          src_v0, dst_v0, rows_v0, src_v1, dst_v1, rows_v1,
          st8_v, sem_g0, sem_g1, sem_s1):
        c = lax.axis_index("c")
        s = lax.axis_index("s")
        wid = s * NC + c

        @pl.when(s < 15)
        def _():
            pltpu.sync_copy(z8_hbm.at[pl.ds(0, SL_A)], st8_v.at[pl.ds(0, SL_A)])
            pltpu.sync_copy(st8_v.at[pl.ds(0, SL_A)],
                            agg_sh.at[pl.ds(s * SL_A, SL_A)])

        @pl.when(s == 15)
        def _():
            pltpu.sync_copy(z8_hbm, st8_v)
            pltpu.sync_copy(st8_v, agg_sh.at[pl.ds(15 * SL_A, SL_B)])

        plsc.subcore_barrier()

        base = wid * EPW
        srcb = [src_v0, src_v1]
        dstb = [dst_v0, dst_v1]
        rowb = [rows_v0, rows_v1]
        semg = [sem_g0, sem_g1]

        def load_idx(j):
            off = pl.multiple_of(base + j * CH, 8)
            pltpu.sync_copy(src_hbm.at[pl.ds(off, CH)], srcb[j % 2])
            pltpu.sync_copy(dst_hbm.at[pl.ds(off, CH)], dstb[j % 2])

        load_idx(0)
        gathers = {0: pltpu.async_copy(y2_hbm.at[srcb[0]], rowb[0], sem_g0)}
        scatters = {}
        for j in range(NCH):
            if j >= 1:
                scatters.pop(j - 1).wait()
            if j + 1 < NCH:
                load_idx(j + 1)
            gathers.pop(j).wait()
            scatters[j] = pltpu.async_copy(
                rowb[j % 2], agg_sh.at[dstb[j % 2]], sem_s1, add=True)
            if j + 1 < NCH:
                gathers[j + 1] = pltpu.async_copy(
                    y2_hbm.at[srcb[(j + 1) % 2]], rowb[(j + 1) % 2],
                    semg[(j + 1) % 2])
        scatters.pop(NCH - 1).wait()
        plsc.subcore_barrier()

        @pl.when(s < 15)
        def _():
            n0 = s * SL_A
            pltpu.sync_copy(agg_sh.at[pl.ds(n0, SL_A)], st8_v.at[pl.ds(0, SL_A)])
            pltpu.sync_copy(st8_v.at[pl.ds(0, SL_A)],
                            agg_out.at[pl.ds(c * N_NODES + n0, SL_A)])

        @pl.when(s == 15)
        def _():
            n0 = 15 * SL_A
            pltpu.sync_copy(agg_sh.at[pl.ds(n0, SL_B)], st8_v)
            pltpu.sync_copy(st8_v, agg_out.at[pl.ds(c * N_NODES + n0, SL_B)])

    return k(src, dst, y2b, zeros8)


def _tc_layer_mid(aggp, degp, p1, b1l, w2l, w2r, b2l):
    def body(aggp_ref, degp_ref, p1_ref, b1l_ref, w2l_ref, w2r_ref, b2l_ref,
             y2_ref, p2b_ref, degc_ref):
        agg = aggp_ref[0:N_NODES, :] + aggp_ref[N_NODES:2 * N_NODES, :]
        deg = degp_ref[0:N_NODES, 0:1] + degp_ref[N_NODES:2 * N_NODES, 0:1]
        degc = jnp.maximum(deg, 1.0)
        h = jnp.maximum(agg / degc + b1l_ref[...] + p1_ref[...], 0.0)
        y2 = jnp.sum(h * w2l_ref[...], axis=1, keepdims=True)
        y2_ref[...] = jnp.broadcast_to(y2, (N_NODES, W8))
        p2b_ref[...] = jnp.sum(h * w2r_ref[...], axis=1, keepdims=True) + b2l_ref[...]
        degc_ref[...] = degc

    return pl.pallas_call(
        body,
        out_shape=[
            jax.ShapeDtypeStruct((N_NODES, W8), jnp.float32),
            jax.ShapeDtypeStruct((N_NODES, 1), jnp.float32),
            jax.ShapeDtypeStruct((N_NODES, 1), jnp.float32),
        ],
    )(aggp, degp, p1, b1l, w2l, w2r, b2l)


def _tc_final(agg2p, degc, p2b):
    def body(a_ref, d_ref, p_ref, o_ref):
        a = a_ref[0:N_NODES, 0:1] + a_ref[N_NODES:2 * N_NODES, 0:1]
        o_ref[...] = a / d_ref[...] + p_ref[...]

    return pl.pallas_call(
        body,
        out_shape=jax.ShapeDtypeStruct((N_NODES, 1), jnp.float32),
    )(agg2p, degc, p2b)


def kernel(x, edge_index, W1l, b1l, W1r, W2l, b2l, W2r):
    src = edge_index[0].astype(jnp.int32)
    dst = edge_index[1].astype(jnp.int32)

    zeros16 = jnp.zeros((SL_B, D_HID), jnp.float32)
    zeros8 = jnp.zeros((SL_B, W8), jnp.float32)
    ones8 = jnp.ones((CH, W8), jnp.float32)

    y1, p1 = _tc_linear2(x, W1l.T, W1r.T)
    aggp, degp = _sc_aggregate1(src, dst, y1, zeros16, zeros8, ones8)
    y2b, p2b, degc = _tc_layer_mid(
        aggp, degp, p1, b1l.reshape(1, D_HID), W2l, W2r, b2l.reshape(1, 1))
    agg2p = _sc_aggregate2(src, dst, y2b, zeros8)
    out = _tc_final(agg2p, degc, p2b)
    return out


# speedup vs baseline: 1.2381x; 1.1201x over previous
import functools

import jax
import jax.numpy as jnp
from jax import lax
from jax.experimental import pallas as pl
from jax.experimental.pallas import tpu as pltpu
from jax.experimental.pallas import tpu_sc as plsc

N_NODES = 10000
N_EDGES = 320000
D_IN = 128
D_HID = 16
W8 = 8

NC = 2
NS = 16
NW = NC * NS
EPW = N_EDGES // NW
CH = 2000
NCH = EPW // CH
SL_A = 624
SL_B = N_NODES - 15 * SL_A


def _tc_linear2(x, wl, wr):
    def body(x_ref, wl_ref, wr_ref, y_ref, p_ref):
        xv = x_ref[...]
        y_ref[...] = jnp.dot(xv, wl_ref[...], preferred_element_type=jnp.float32)
        p_ref[...] = jnp.dot(xv, wr_ref[...], preferred_element_type=jnp.float32)

    n = x.shape[0]
    return pl.pallas_call(
        body,
        out_shape=[
            jax.ShapeDtypeStruct((n, wl.shape[1]), jnp.float32),
            jax.ShapeDtypeStruct((n, wr.shape[1]), jnp.float32),
        ],
    )(x, wl, wr)


def _sc_aggregate1(ei, y1, zeros16, zeros8, ones8):
    mesh = plsc.VectorSubcoreMesh(core_axis_name="c", subcore_axis_name="s")

    @functools.partial(
        pl.kernel,
        out_type=[
            jax.ShapeDtypeStruct((NC * N_NODES, D_HID), jnp.float32),
            jax.ShapeDtypeStruct((NC * N_NODES, W8), jnp.float32),
        ],
        mesh=mesh,
        compiler_params=pltpu.CompilerParams(use_tc_tiling_on_sc=False),
        scratch_types=[
            pltpu.VMEM_SHARED((N_NODES, D_HID), jnp.float32),
            pltpu.VMEM_SHARED((N_NODES, W8), jnp.float32),
            pltpu.VMEM((CH,), jnp.int32),
            pltpu.VMEM((CH,), jnp.int32),
            pltpu.VMEM((CH, D_HID), jnp.float32),
            pltpu.VMEM((CH,), jnp.int32),
            pltpu.VMEM((CH,), jnp.int32),
            pltpu.VMEM((CH, D_HID), jnp.float32),
            pltpu.VMEM((CH, W8), jnp.float32),
            pltpu.VMEM((SL_B, D_HID), jnp.float32),
            pltpu.VMEM((SL_B, W8), jnp.float32),
            pltpu.SemaphoreType.DMA,
            pltpu.SemaphoreType.DMA,
            pltpu.SemaphoreType.DMA,
            pltpu.SemaphoreType.DMA,
        ],
    )
    def k(ei_hbm, y1_hbm, z16_hbm, z8_hbm, ones_hbm,
          agg_out, deg_out, agg_sh, deg_sh,
          src_v0, dst_v0, rows_v0, src_v1, dst_v1, rows_v1,
          ones_v, st16_v, st8_v, sem_g0, sem_g1, sem_s1, sem_s2):
        c = lax.axis_index("c")
        s = lax.axis_index("s")
        wid = s * NC + c

        pltpu.sync_copy(ones_hbm, ones_v)

        @pl.when(s < 15)
        def _():
            n0 = s * SL_A
            pltpu.sync_copy(z16_hbm.at[pl.ds(0, SL_A)], st16_v.at[pl.ds(0, SL_A)])
            pltpu.sync_copy(st16_v.at[pl.ds(0, SL_A)], agg_sh.at[pl.ds(n0, SL_A)])
            pltpu.sync_copy(z8_hbm.at[pl.ds(0, SL_A)], st8_v.at[pl.ds(0, SL_A)])
            pltpu.sync_copy(st8_v.at[pl.ds(0, SL_A)], deg_sh.at[pl.ds(n0, SL_A)])

        @pl.when(s == 15)
        def _():
            pltpu.sync_copy(z16_hbm, st16_v)
            pltpu.sync_copy(st16_v, agg_sh.at[pl.ds(15 * SL_A, SL_B)])
            pltpu.sync_copy(z8_hbm, st8_v)
            pltpu.sync_copy(st8_v, deg_sh.at[pl.ds(15 * SL_A, SL_B)])

        plsc.subcore_barrier()

        base = wid * EPW
        srcb = [src_v0, src_v1]
        dstb = [dst_v0, dst_v1]
        rowb = [rows_v0, rows_v1]
        semg = [sem_g0, sem_g1]

        def load_idx(j):
            off = pl.multiple_of(base + j * CH, 8)
            pltpu.sync_copy(ei_hbm.at[0, pl.ds(off, CH)], srcb[j % 2])
            pltpu.sync_copy(ei_hbm.at[1, pl.ds(off, CH)], dstb[j % 2])

        load_idx(0)
        gathers = {0: pltpu.async_copy(y1_hbm.at[srcb[0]], rowb[0], sem_g0)}
        scatters = {}
        for j in range(NCH):
            if j >= 1:
                for d in scatters.pop(j - 1):
                    d.wait()
            if j + 1 < NCH:
                load_idx(j + 1)
            gathers.pop(j).wait()
            scatters[j] = (
                pltpu.async_copy(rowb[j % 2], agg_sh.at[dstb[j % 2]], sem_s1,
                                 add=True),
                pltpu.async_copy(ones_v, deg_sh.at[dstb[j % 2]], sem_s2,
                                 add=True),
            )
            if j + 1 < NCH:
                gathers[j + 1] = pltpu.async_copy(
                    y1_hbm.at[srcb[(j + 1) % 2]], rowb[(j + 1) % 2],
                    semg[(j + 1) % 2])
        for d in scatters.pop(NCH - 1):
            d.wait()
        plsc.subcore_barrier()

        @pl.when(s < 15)
        def _():
            n0 = s * SL_A
            r0 = c * N_NODES + n0
            pltpu.sync_copy(agg_sh.at[pl.ds(n0, SL_A)], st16_v.at[pl.ds(0, SL_A)])
            pltpu.sync_copy(st16_v.at[pl.ds(0, SL_A)], agg_out.at[pl.ds(r0, SL_A)])
            pltpu.sync_copy(deg_sh.at[pl.ds(n0, SL_A)], st8_v.at[pl.ds(0, SL_A)])
            pltpu.sync_copy(st8_v.at[pl.ds(0, SL_A)], deg_out.at[pl.ds(r0, SL_A)])

        @pl.when(s == 15)
        def _():
            n0 = 15 * SL_A
            r0 = c * N_NODES + n0
            pltpu.sync_copy(agg_sh.at[pl.ds(n0, SL_B)], st16_v)
            pltpu.sync_copy(st16_v, agg_out.at[pl.ds(r0, SL_B)])
            pltpu.sync_copy(deg_sh.at[pl.ds(n0, SL_B)], st8_v)
            pltpu.sync_copy(st8_v, deg_out.at[pl.ds(r0, SL_B)])

    return k(ei, y1, zeros16, zeros8, ones8)


def _sc_aggregate2(ei, y2b, zeros8):
    mesh = plsc.VectorSubcoreMesh(core_axis_name="c", subcore_axis_name="s")

    @functools.partial(
        pl.kernel,
        out_type=jax.ShapeDtypeStruct((NC * N_NODES, W8), jnp.float32),
        mesh=mesh,
        compiler_params=pltpu.CompilerParams(use_tc_tiling_on_sc=False),
        scratch_types=[
            pltpu.VMEM_SHARED((N_NODES, W8), jnp.float32),
            pltpu.VMEM((CH,), jnp.int32),
            pltpu.VMEM((CH,), jnp.int32),
            pltpu.VMEM((CH, W8), jnp.float32),
            pltpu.VMEM((CH,), jnp.int32),
            pltpu.VMEM((CH,), jnp.int32),
            pltpu.VMEM((CH, W8), jnp.float32),
            pltpu.VMEM((SL_B, W8), jnp.float32),
            pltpu.SemaphoreType.DMA,
            pltpu.SemaphoreType.DMA,
            pltpu.SemaphoreType.DMA,
        ],
    )
    def k(ei_hbm, y2_hbm, z8_hbm, agg_out, agg_sh,
          src_v0, dst_v0, rows_v0, src_v1, dst_v1, rows_v1,
          st8_v, sem_g0, sem_g1, sem_s1):
        c = lax.axis_index("c")
        s = lax.axis_index("s")
        wid = s * NC + c

        @pl.when(s < 15)
        def _():
            pltpu.sync_copy(z8_hbm.at[pl.ds(0, SL_A)], st8_v.at[pl.ds(0, SL_A)])
            pltpu.sync_copy(st8_v.at[pl.ds(0, SL_A)],
                            agg_sh.at[pl.ds(s * SL_A, SL_A)])

        @pl.when(s == 15)
        def _():
            pltpu.sync_copy(z8_hbm, st8_v)
            pltpu.sync_copy(st8_v, agg_sh.at[pl.ds(15 * SL_A, SL_B)])

        plsc.subcore_barrier()

        base = wid * EPW
        srcb = [src_v0, src_v1]
        dstb = [dst_v0, dst_v1]
        rowb = [rows_v0, rows_v1]
        semg = [sem_g0, sem_g1]

        def load_idx(j):
            off = pl.multiple_of(base + j * CH, 8)
            pltpu.sync_copy(ei_hbm.at[0, pl.ds(off, CH)], srcb[j % 2])
            pltpu.sync_copy(ei_hbm.at[1, pl.ds(off, CH)], dstb[j % 2])

        load_idx(0)
        gathers = {0: pltpu.async_copy(y2_hbm.at[srcb[0]], rowb[0], sem_g0)}
        scatters = {}
        for j in range(NCH):
            if j >= 1:
                scatters.pop(j - 1).wait()
            if j + 1 < NCH:
                load_idx(j + 1)
            gathers.pop(j).wait()
            scatters[j] = pltpu.async_copy(
                rowb[j % 2], agg_sh.at[dstb[j % 2]], sem_s1, add=True)
            if j + 1 < NCH:
                gathers[j + 1] = pltpu.async_copy(
                    y2_hbm.at[srcb[(j + 1) % 2]], rowb[(j + 1) % 2],
                    semg[(j + 1) % 2])
        scatters.pop(NCH - 1).wait()
        plsc.subcore_barrier()

        @pl.when(s < 15)
        def _():
            n0 = s * SL_A
            pltpu.sync_copy(agg_sh.at[pl.ds(n0, SL_A)], st8_v.at[pl.ds(0, SL_A)])
            pltpu.sync_copy(st8_v.at[pl.ds(0, SL_A)],
                            agg_out.at[pl.ds(c * N_NODES + n0, SL_A)])

        @pl.when(s == 15)
        def _():
            n0 = 15 * SL_A
            pltpu.sync_copy(agg_sh.at[pl.ds(n0, SL_B)], st8_v)
            pltpu.sync_copy(st8_v, agg_out.at[pl.ds(c * N_NODES + n0, SL_B)])

    return k(ei, y2b, zeros8)


def _tc_layer_mid(aggp, degp, p1, b1l, w2l, w2r, b2l):
    def body(aggp_ref, degp_ref, p1_ref, b1l_ref, w2l_ref, w2r_ref, b2l_ref,
             y2_ref, p2b_ref, degc_ref):
        agg = aggp_ref[0:N_NODES, :] + aggp_ref[N_NODES:2 * N_NODES, :]
        deg = degp_ref[0:N_NODES, 0:1] + degp_ref[N_NODES:2 * N_NODES, 0:1]
        dinv = 1.0 / jnp.maximum(deg, 1.0)
        h = jnp.maximum(agg * dinv + b1l_ref[...] + p1_ref[...], 0.0)
        y2 = jnp.sum(h * w2l_ref[...], axis=1, keepdims=True)
        y2_ref[...] = jnp.broadcast_to(y2, (N_NODES, W8))
        p2b_ref[...] = jnp.sum(h * w2r_ref[...], axis=1, keepdims=True) + b2l_ref[...]
        degc_ref[...] = dinv

    return pl.pallas_call(
        body,
        out_shape=[
            jax.ShapeDtypeStruct((N_NODES, W8), jnp.float32),
            jax.ShapeDtypeStruct((N_NODES, 1), jnp.float32),
            jax.ShapeDtypeStruct((N_NODES, 1), jnp.float32),
        ],
    )(aggp, degp, p1, b1l, w2l, w2r, b2l)


def _tc_final(agg2p, degc, p2b):
    def body(a_ref, d_ref, p_ref, o_ref):
        a = a_ref[0:N_NODES, 0:1] + a_ref[N_NODES:2 * N_NODES, 0:1]
        o_ref[...] = a * d_ref[...] + p_ref[...]

    return pl.pallas_call(
        body,
        out_shape=jax.ShapeDtypeStruct((N_NODES, 1), jnp.float32),
    )(agg2p, degc, p2b)


def kernel(x, edge_index, W1l, b1l, W1r, W2l, b2l, W2r):
    ei = edge_index.astype(jnp.int32)

    zeros16 = jnp.zeros((SL_B, D_HID), jnp.float32)
    zeros8 = jnp.zeros((SL_B, W8), jnp.float32)
    ones8 = jnp.ones((CH, W8), jnp.float32)

    y1, p1 = _tc_linear2(x, W1l.T, W1r.T)
    aggp, degp = _sc_aggregate1(ei, y1, zeros16, zeros8, ones8)
    y2b, p2b, degc = _tc_layer_mid(
        aggp, degp, p1, b1l.reshape(1, D_HID), W2l, W2r, b2l.reshape(1, 1))
    agg2p = _sc_aggregate2(ei, y2b, zeros8)
    out = _tc_final(agg2p, degc, p2b)
    return out
